# baseline (speedup 1.0000x reference)
import functools
import jax
import jax.numpy as jnp
from jax import lax
from jax.experimental import pallas as pl
from jax.experimental.pallas import tpu as pltpu
from jax.experimental.pallas import tpu_sc as plsc

V = 8192
D = 1024
B = 4 * 8192
NC = 2
NS = 16
NW = NC * NS
BPW = B // NW
K = 32
NCHUNK = BPW // K


NBUF = 3


def _sc_body(idx_hbm, table_hbm, out_hbm, idx_v,
             r0, r1, r2, g0, g1, g2, w0, w1, w2):
    wid = lax.axis_index("s") * NC + lax.axis_index("c")
    pltpu.sync_copy(idx_hbm.at[wid], idx_v)

    bufs = (r0, r1, r2)
    gsems = (g0, g1, g2)
    wsems = (w0, w1, w2)

    gathers = [None] * NCHUNK
    writes = [None] * NCHUNK
    for j in range(min(NBUF, NCHUNK)):
        gathers[j] = pltpu.async_copy(
            table_hbm.at[idx_v.at[j]], bufs[j % NBUF], gsems[j % NBUF])
    for j in range(NCHUNK):
        b = j % NBUF
        gathers[j].wait()
        writes[j] = pltpu.async_copy(bufs[b], out_hbm.at[wid, j], wsems[b])
        if j >= 1:
            writes[j - 1].wait()
            nj = j + 2
            if nj < NCHUNK and gathers[nj] is None:
                nb = nj % NBUF
                gathers[nj] = pltpu.async_copy(
                    table_hbm.at[idx_v.at[nj]], bufs[nb], gsems[nb])
    writes[NCHUNK - 1].wait()


@jax.jit
def _sc_gather(idx3, table):
    mesh = plsc.VectorSubcoreMesh(core_axis_name="c", subcore_axis_name="s")
    run = pl.kernel(
        _sc_body,
        mesh=mesh,
        out_type=jax.ShapeDtypeStruct((NW, NCHUNK, K, D), jnp.float32),
        scratch_types=(
            [pltpu.VMEM((NCHUNK, K), jnp.int32)]
            + [pltpu.VMEM((K, D), jnp.float32)] * NBUF
            + [pltpu.SemaphoreType.DMA] * (2 * NBUF)
        ),
    )
    return run(idx3, table)


def kernel(X, pos_embed_weight):
    idx3 = X.reshape(NW, NCHUNK, K).astype(jnp.int32)
    out = _sc_gather(idx3, pos_embed_weight)
    return out.reshape(X.shape + (D,))
